# baseline (speedup 1.0000x reference)
import functools
import math

import jax
import jax.numpy as jnp
from jax import lax
from jax.experimental import pallas as pl
from jax.experimental.pallas import tpu as pltpu
from jax.experimental.pallas import tpu_sc as plsc

K = 2
NL = 16


def _router_kernel(x_ref, w_ref, bsw_ref, logits_ref, acc_ref):
    i = pl.program_id(0)

    @pl.when(i == 0)
    def _init():
        acc_ref[...] = jnp.zeros_like(acc_ref)

    xb = x_ref[...]
    xb2 = xb.reshape(xb.shape[0], xb.shape[1] * xb.shape[2])
    acc_ref[...] += jax.lax.dot_general(
        xb2, w_ref[...],
        (((1,), (1,)), ((), ())),
        preferred_element_type=jnp.float32,
    )

    @pl.when(i == pl.num_programs(0) - 1)
    def _finish():
        b, e = acc_ref.shape
        lt = (acc_ref[...] + bsw_ref[...]).T
        pad = jnp.full((e, NL - b), -1e30, jnp.float32)
        logits_ref[...] = jnp.concatenate([lt, pad], axis=1)


def _sc_route_body(logits_hbm, idx_hbm, gval_hbm, lg_v, idx_v, gv_v):
    first = (lax.axis_index("c") == 0) & (lax.axis_index("s") == 0)

    @pl.when(first)
    def _():
        pltpu.sync_copy(logits_hbm, lg_v)
        ne = lg_v.shape[0]
        rows = [lg_v[e, :] for e in range(ne)]
        m1 = rows[0]
        i1 = jnp.zeros((NL,), jnp.int32)
        for e in range(1, ne):
            ev = jnp.full((NL,), e, jnp.int32)
            better = rows[e] > m1
            m1 = jnp.where(better, rows[e], m1)
            i1 = jnp.where(better, ev, i1)
        m2 = jnp.full((NL,), -jnp.inf, jnp.float32)
        i2 = jnp.zeros((NL,), jnp.int32)
        for e in range(ne):
            ev = jnp.full((NL,), e, jnp.int32)
            cand = (ev != i1) & (rows[e] > m2)
            m2 = jnp.where(cand, rows[e], m2)
            i2 = jnp.where(cand, ev, i2)
        tot = jnp.zeros((NL,), jnp.float32)
        for e in range(ne):
            tot = tot + jnp.exp(rows[e] - m1)
        g1 = jnp.ones((NL,), jnp.float32) / tot
        g2 = jnp.exp(m2 - m1) / tot
        idx_v[0, :] = i1
        idx_v[1, :] = i2
        gv_v[0, :] = g1
        gv_v[1, :] = g2
        pltpu.sync_copy(idx_v, idx_hbm)
        pltpu.sync_copy(gv_v, gval_hbm)


def _one_expert(x_ref, w1_ref, b1_ref, w2_ref, g):
    xb = x_ref[0] * g
    h = jax.lax.dot_general(
        xb, w1_ref[0], (((1,), (1,)), ((), ())),
        preferred_element_type=jnp.float32,
    ) + b1_ref[0]
    a = 0.5 * h * (1.0 + jax.lax.erf(h * (1.0 / math.sqrt(2.0))))
    return jax.lax.dot_general(
        a, w2_ref[...], (((1,), (1,)), ((), ())),
        preferred_element_type=jnp.float32,
    )


def _expert_kernel(idx_s, gval_s, x0_ref, x1_ref, w1a_ref, w1b_ref,
                   b1a_ref, b1b_ref, w2a_ref, w2b_ref, b2_ref, o_ref):
    del idx_s
    b = pl.program_id(0)
    y0 = _one_expert(x0_ref, w1a_ref, b1a_ref, w2a_ref, gval_s[0, b])
    y1 = _one_expert(x1_ref, w1b_ref, b1b_ref, w2b_ref, gval_s[1, b])
    o_ref[0] = (y0 + y1) + b2_ref[...]


@jax.jit
def kernel(x, Wsw, bsw, W1, b1, W2, b2):
    B, S, D = x.shape
    E, SH, SD = W1.shape
    H = W2.shape[1]

    NC = 16
    Ss = S // NC
    C = Ss * D
    logits = pl.pallas_call(
        _router_kernel,
        grid=(NC,),
        in_specs=[
            pl.BlockSpec((B, Ss, D), lambda i: (0, i, 0)),
            pl.BlockSpec((E, C), lambda i: (0, i)),
            pl.BlockSpec((1, E), lambda i: (0, 0)),
        ],
        out_specs=pl.BlockSpec((E, NL), lambda i: (0, 0)),
        out_shape=jax.ShapeDtypeStruct((E, NL), jnp.float32),
        scratch_shapes=[pltpu.VMEM((B, E), jnp.float32)],
        compiler_params=pltpu.CompilerParams(
            dimension_semantics=("arbitrary",),
        ),
    )(x, Wsw, bsw.reshape(1, E))

    mesh = plsc.VectorSubcoreMesh(core_axis_name="c", subcore_axis_name="s")
    idx, gval = pl.kernel(
        _sc_route_body,
        mesh=mesh,
        out_type=[
            jax.ShapeDtypeStruct((K, NL), jnp.int32),
            jax.ShapeDtypeStruct((K, NL), jnp.float32),
        ],
        scratch_types=[
            pltpu.VMEM((E, NL), jnp.float32),
            pltpu.VMEM((K, NL), jnp.int32),
            pltpu.VMEM((K, NL), jnp.float32),
        ],
    )(logits)

    St = 512
    S_TILES = S // St
    b1r = b1.reshape(E, 1, SH)
    b2r = b2.reshape(1, D)
    grid_spec = pltpu.PrefetchScalarGridSpec(
        num_scalar_prefetch=2,
        grid=(B, S_TILES),
        in_specs=[
            pl.BlockSpec((1, St, SD), lambda b, s, idx_s, gv: (b, s, idx_s[0, b])),
            pl.BlockSpec((1, St, SD), lambda b, s, idx_s, gv: (b, s, idx_s[1, b])),
            pl.BlockSpec((1, SH, SD), lambda b, s, idx_s, gv: (idx_s[0, b], 0, 0)),
            pl.BlockSpec((1, SH, SD), lambda b, s, idx_s, gv: (idx_s[1, b], 0, 0)),
            pl.BlockSpec((1, 1, SH), lambda b, s, idx_s, gv: (idx_s[0, b], 0, 0)),
            pl.BlockSpec((1, 1, SH), lambda b, s, idx_s, gv: (idx_s[1, b], 0, 0)),
            pl.BlockSpec((D, SH), lambda b, s, idx_s, gv: (0, idx_s[0, b])),
            pl.BlockSpec((D, SH), lambda b, s, idx_s, gv: (0, idx_s[1, b])),
            pl.BlockSpec((1, D), lambda b, s, idx_s, gv: (0, 0)),
        ],
        out_specs=pl.BlockSpec((1, St, D), lambda b, s, idx_s, gv: (b, s, 0)),
    )
    y = pl.pallas_call(
        _expert_kernel,
        grid_spec=grid_spec,
        out_shape=jax.ShapeDtypeStruct((B, S, D), jnp.float32),
        compiler_params=pltpu.CompilerParams(
            dimension_semantics=("parallel", "parallel"),
        ),
    )(idx, gval, x, x, W1, W1, b1r, b1r, W2, W2, b2r)
    return y
